# baseline (speedup 1.0000x reference)
import functools

import jax
import jax.numpy as jnp
from jax import lax
from jax.experimental import pallas as pl
from jax.experimental.pallas import tpu as pltpu
from jax.experimental.pallas import tpu_sc as plsc

_SCALE = 30.0
_MARGIN = 0.4
_B = 1024
_C = 100000
_LN2 = 0.6931471805599453
_K2 = _SCALE / _LN2
_NEG = -1e30

_WSC = 364
_CSC = _WSC * 128
_WIN = 14
_NWIN = _WSC // _WIN
_WINC = _WIN * 128


_NC = 2
_NS = 16
_NW = _NC * _NS
_BANDS_PER_TEC = 128 // _NW


def _sc_body(x_hbm, yrep_hbm, m_hbm, s_hbm, t_hbm,
             xbuf, ytile, m_acc, s_acc, t_acc, m_old):
    wid = lax.axis_index("s") * _NC + lax.axis_index("c")
    iota = lax.broadcasted_iota(jnp.int32, (16,), 0)
    neg = jnp.full((16,), _NEG, jnp.float32)
    zero = jnp.zeros((16,), jnp.float32)

    for bi in range(_BANDS_PER_TEC):
        band = wid * _BANDS_PER_TEC + bi
        r0 = pl.multiple_of(band * 8, 8)
        pltpu.sync_copy(yrep_hbm.at[pl.ds(r0, 8)], ytile)
        for r in range(8):
            for q in range(8):
                m_acc[r, pl.ds(q * 16, 16)] = neg
                s_acc[r, pl.ds(q * 16, 16)] = zero
                t_acc[r, pl.ds(q * 16, 16)] = zero

        def window(w, carry):
            c0 = pl.multiple_of(w * _WINC, 128)
            pltpu.sync_copy(x_hbm.at[pl.ds(r0, 8), pl.ds(c0, _WINC)], xbuf)

            def tile_pass1(tl, carry):
                for r in range(8):
                    for q in range(8):
                        v = xbuf[r, pl.ds(tl * 128 + q * 16, 16)]
                        sl = pl.ds(q * 16, 16)
                        m_old[r, sl] = m_acc[r, sl]
                        m_acc[r, sl] = jnp.maximum(m_acc[r, sl], v)
                return carry

            def tile_pass1_rest(tl, carry):
                for r in range(8):
                    for q in range(8):
                        v = xbuf[r, pl.ds(tl * 128 + q * 16, 16)]
                        sl = pl.ds(q * 16, 16)
                        m_acc[r, sl] = jnp.maximum(m_acc[r, sl], v)
                return carry

            tile_pass1(0, 0)
            lax.fori_loop(1, _WIN, tile_pass1_rest, 0, unroll=False)

            for r in range(8):
                for q in range(8):
                    sl = pl.ds(q * 16, 16)
                    s_acc[r, sl] = s_acc[r, sl] * jnp.exp(
                        (m_old[r, sl] - m_acc[r, sl]) * _SCALE)

            def tile_pass2(tl, carry):
                colb = c0 + tl * 128
                for r in range(8):
                    yb = ytile[r, pl.ds(0, 16)]
                    for q in range(8):
                        v = xbuf[r, pl.ds(tl * 128 + q * 16, 16)]
                        sl = pl.ds(q * 16, 16)
                        s_acc[r, sl] = s_acc[r, sl] + jnp.exp(
                            (v - m_acc[r, sl]) * _SCALE)
                        colv = colb + q * 16 + iota
                        t_acc[r, sl] = t_acc[r, sl] + jnp.where(
                            colv == yb, v, 0.0)
                return carry

            lax.fori_loop(0, _WIN, tile_pass2, 0, unroll=False)
            return carry

        lax.fori_loop(0, _NWIN, window, 0, unroll=False)

        pltpu.sync_copy(m_acc, m_hbm.at[pl.ds(r0, 8)])
        pltpu.sync_copy(s_acc, s_hbm.at[pl.ds(r0, 8)])
        pltpu.sync_copy(t_acc, t_hbm.at[pl.ds(r0, 8)])


@functools.cache
def _sc_region():
    mesh = plsc.VectorSubcoreMesh(
        core_axis_name="c", subcore_axis_name="s",
        num_cores=_NC, num_subcores=_NS)
    return pl.kernel(
        _sc_body,
        out_type=(
            jax.ShapeDtypeStruct((_B, 128), jnp.float32),
            jax.ShapeDtypeStruct((_B, 128), jnp.float32),
            jax.ShapeDtypeStruct((_B, 128), jnp.float32),
        ),
        mesh=mesh,
        scratch_types=[
            pltpu.VMEM((8, _WINC), jnp.float32),
            pltpu.VMEM((8, 128), jnp.int32),
            pltpu.VMEM((8, 128), jnp.float32),
            pltpu.VMEM((8, 128), jnp.float32),
            pltpu.VMEM((8, 128), jnp.float32),
            pltpu.VMEM((8, 128), jnp.float32),
        ],
        compiler_params=pltpu.CompilerParams(use_tc_tiling_on_sc=True),
    )


_BC = 3584
_K0 = _CSC // _BC
_K = (_C - _CSC + _BC - 1) // _BC


def _tc_body(y_ref, x_ref, m_out, s_out, t_out, m_ref, s_ref, t_ref):
    k = pl.program_id(0)
    nk = pl.num_programs(0)

    @pl.when(k == 0)
    def _init():
        m_ref[...] = jnp.full_like(m_ref, _NEG)
        s_ref[...] = jnp.zeros_like(s_ref)
        t_ref[...] = jnp.zeros_like(t_ref)

    def step(xb, col):
        y2 = y_ref[...]
        t_ref[...] += jnp.sum(jnp.where(col == y2, xb, 0.0), axis=1,
                              keepdims=True)
        m_old = m_ref[...]
        m_new = jnp.maximum(m_old, jnp.max(xb, axis=1, keepdims=True))
        s_ref[...] = (s_ref[...] * jnp.exp2((m_old - m_new) * _K2)
                      + jnp.sum(jnp.exp2((xb - m_new) * _K2), axis=1,
                                keepdims=True))
        m_ref[...] = m_new

    def colidx(k):
        return (_K0 + k) * _BC + lax.broadcasted_iota(jnp.int32, (_B, _BC), 1)

    @pl.when(k < nk - 1)
    def _hot():
        step(x_ref[...], colidx(k))

    @pl.when(k == nk - 1)
    def _tail():
        col = colidx(k)
        step(jnp.where(col < _C, x_ref[...], _NEG), col)
        m_out[...] = m_ref[...]
        s_out[...] = s_ref[...]
        t_out[...] = t_ref[...]


_tc_lse = pl.pallas_call(
    _tc_body,
    grid=(_K,),
    in_specs=[
        pl.BlockSpec((_B, 1), lambda k: (0, 0)),
        pl.BlockSpec((_B, _BC), lambda k: (0, _K0 + k)),
    ],
    out_specs=[
        pl.BlockSpec((_B, 1), lambda k: (0, 0)),
        pl.BlockSpec((_B, 1), lambda k: (0, 0)),
        pl.BlockSpec((_B, 1), lambda k: (0, 0)),
    ],
    out_shape=[
        jax.ShapeDtypeStruct((_B, 1), jnp.float32),
        jax.ShapeDtypeStruct((_B, 1), jnp.float32),
        jax.ShapeDtypeStruct((_B, 1), jnp.float32),
    ],
    scratch_shapes=[
        pltpu.VMEM((_B, 1), jnp.float32),
        pltpu.VMEM((_B, 1), jnp.float32),
        pltpu.VMEM((_B, 1), jnp.float32),
    ],
)


def _combine_body(m0_ref, s0_ref, t0_ref, m1_ref, s1_ref, t1_ref, out_ref):
    m0 = m0_ref[...]
    s0 = s0_ref[...]
    m1 = m1_ref[...]
    s1 = s1_ref[...]
    m = jnp.maximum(m0, jnp.max(m1, axis=1, keepdims=True))
    s = (s0 * jnp.exp2((m0 - m) * _K2)
         + jnp.sum(s1 * jnp.exp((m1 - m) * _SCALE), axis=1, keepdims=True))
    t = t0_ref[...] + jnp.sum(t1_ref[...], axis=1, keepdims=True)
    zt = (t - m) * _K2
    s_mod = s - jnp.exp2(zt) + jnp.exp2(zt - _SCALE * _MARGIN / _LN2)
    row_loss = (_SCALE * m + jnp.log(s_mod) - _SCALE * t
                + _SCALE * _MARGIN)
    out_ref[0, 0] = jnp.sum(row_loss) * (1.0 / _B)


_combine = pl.pallas_call(
    _combine_body,
    out_specs=pl.BlockSpec(memory_space=pltpu.SMEM),
    out_shape=jax.ShapeDtypeStruct((1, 1), jnp.float32),
)


def kernel(x, y):
    y32 = y.astype(jnp.int32)
    yrep = jnp.broadcast_to(y32[:, None], (_B, 128))
    m1, s1, t1 = _sc_region()(x, yrep)
    m0, s0, t0 = _tc_lse(y32.reshape(_B, 1), x)
    loss = _combine(m0, s0, t0, m1, s1, t1)
    return loss[0, 0]


# speedup vs baseline: 3.1858x; 3.1858x over previous
import functools

import jax
import jax.numpy as jnp
from jax import lax
from jax.experimental import pallas as pl
from jax.experimental.pallas import tpu as pltpu
from jax.experimental.pallas import tpu_sc as plsc

_SCALE = 30.0
_MARGIN = 0.4
_B = 1024
_C = 100000
_LN2 = 0.6931471805599453
_K2 = _SCALE / _LN2
_NEG = -1e30

_WSC = 364
_CSC = _WSC * 128
_WIN = 26
_NWIN = _WSC // _WIN
_WINC = _WIN * 128


_NC = 2
_NS = 16
_NW = _NC * _NS
_BANDS_PER_TEC = 128 // _NW


def _sc_body(x_hbm, yrep_hbm, m_hbm, s_hbm, t_hbm,
             xbuf, ytile, m_acc, s_acc, t_acc):
    wid = lax.axis_index("s") * _NC + lax.axis_index("c")
    iota = lax.broadcasted_iota(jnp.int32, (16,), 0)
    neg = jnp.full((16,), _NEG, jnp.float32)
    zero = jnp.zeros((16,), jnp.float32)

    def band_body(bi, _):
        r0 = pl.multiple_of((wid * _BANDS_PER_TEC + bi) * 8, 8)
        pltpu.sync_copy(yrep_hbm.at[pl.ds(r0, 8)], ytile)
        for r in range(8):
            for q in range(8):
                m_acc[r, pl.ds(q * 16, 16)] = neg
                s_acc[r, pl.ds(q * 16, 16)] = zero
                t_acc[r, pl.ds(q * 16, 16)] = zero

        def window(w, _):
            c0 = pl.multiple_of(w * _WINC, 128)
            pltpu.sync_copy(x_hbm.at[pl.ds(r0, 8), pl.ds(c0, _WINC)], xbuf)
            for r in range(8):
                yb = ytile[r, pl.ds(0, 16)]
                m_prev = tuple(m_acc[r, pl.ds(q * 16, 16)] for q in range(8))

                def pass1(tl, m):
                    return tuple(
                        jnp.maximum(m[q],
                                    xbuf[r, pl.ds(tl * 128 + q * 16, 16)])
                        for q in range(8))

                m_new = lax.fori_loop(0, _WIN, pass1, m_prev)
                s0 = tuple(
                    s_acc[r, pl.ds(q * 16, 16)]
                    * jnp.exp((m_prev[q] - m_new[q]) * _SCALE)
                    for q in range(8))
                t0 = tuple(t_acc[r, pl.ds(q * 16, 16)] for q in range(8))
                ydel = tuple(yb - (q * 16) - iota for q in range(8))

                def pass2(tl, st):
                    s, t = st
                    colb = c0 + tl * 128
                    vs = tuple(xbuf[r, pl.ds(tl * 128 + q * 16, 16)]
                               for q in range(8))
                    s = tuple(
                        s[q] + jnp.exp((vs[q] - m_new[q]) * _SCALE)
                        for q in range(8))
                    t = tuple(
                        t[q] + jnp.where(ydel[q] == colb, vs[q], 0.0)
                        for q in range(8))
                    return (s, t)

                s1, t1 = lax.fori_loop(0, _WIN, pass2, (s0, t0))
                for q in range(8):
                    m_acc[r, pl.ds(q * 16, 16)] = m_new[q]
                    s_acc[r, pl.ds(q * 16, 16)] = s1[q]
                    t_acc[r, pl.ds(q * 16, 16)] = t1[q]
            return 0

        lax.fori_loop(0, _NWIN, window, 0)

        pltpu.sync_copy(m_acc, m_hbm.at[pl.ds(r0, 8)])
        pltpu.sync_copy(s_acc, s_hbm.at[pl.ds(r0, 8)])
        pltpu.sync_copy(t_acc, t_hbm.at[pl.ds(r0, 8)])
        return 0

    lax.fori_loop(0, _BANDS_PER_TEC, band_body, 0)


@functools.cache
def _sc_region():
    mesh = plsc.VectorSubcoreMesh(
        core_axis_name="c", subcore_axis_name="s",
        num_cores=_NC, num_subcores=_NS)
    return pl.kernel(
        _sc_body,
        out_type=(
            jax.ShapeDtypeStruct((_B, 128), jnp.float32),
            jax.ShapeDtypeStruct((_B, 128), jnp.float32),
            jax.ShapeDtypeStruct((_B, 128), jnp.float32),
        ),
        mesh=mesh,
        scratch_types=[
            pltpu.VMEM((8, _WINC), jnp.float32),
            pltpu.VMEM((8, 128), jnp.int32),
            pltpu.VMEM((8, 128), jnp.float32),
            pltpu.VMEM((8, 128), jnp.float32),
            pltpu.VMEM((8, 128), jnp.float32),
        ],
        compiler_params=pltpu.CompilerParams(use_tc_tiling_on_sc=True),
    )


_BC = 3584
_K0 = _CSC // _BC
_K = (_C - _CSC + _BC - 1) // _BC


def _tc_body(y_ref, x_ref, m_out, s_out, t_out, m_ref, s_ref, t_ref):
    k = pl.program_id(0)
    nk = pl.num_programs(0)

    @pl.when(k == 0)
    def _init():
        m_ref[...] = jnp.full_like(m_ref, _NEG)
        s_ref[...] = jnp.zeros_like(s_ref)
        t_ref[...] = jnp.zeros_like(t_ref)

    def step(xb, col):
        y2 = y_ref[...]
        t_ref[...] += jnp.sum(jnp.where(col == y2, xb, 0.0), axis=1,
                              keepdims=True)
        m_old = m_ref[...]
        m_new = jnp.maximum(m_old, jnp.max(xb, axis=1, keepdims=True))
        s_ref[...] = (s_ref[...] * jnp.exp2((m_old - m_new) * _K2)
                      + jnp.sum(jnp.exp2((xb - m_new) * _K2), axis=1,
                                keepdims=True))
        m_ref[...] = m_new

    def colidx(k):
        return (_K0 + k) * _BC + lax.broadcasted_iota(jnp.int32, (_B, _BC), 1)

    @pl.when(k < nk - 1)
    def _hot():
        step(x_ref[...], colidx(k))

    @pl.when(k == nk - 1)
    def _tail():
        col = colidx(k)
        step(jnp.where(col < _C, x_ref[...], _NEG), col)
        m_out[...] = m_ref[...]
        s_out[...] = s_ref[...]
        t_out[...] = t_ref[...]


_tc_lse = pl.pallas_call(
    _tc_body,
    grid=(_K,),
    in_specs=[
        pl.BlockSpec((_B, 1), lambda k: (0, 0)),
        pl.BlockSpec((_B, _BC), lambda k: (0, _K0 + k)),
    ],
    out_specs=[
        pl.BlockSpec((_B, 1), lambda k: (0, 0)),
        pl.BlockSpec((_B, 1), lambda k: (0, 0)),
        pl.BlockSpec((_B, 1), lambda k: (0, 0)),
    ],
    out_shape=[
        jax.ShapeDtypeStruct((_B, 1), jnp.float32),
        jax.ShapeDtypeStruct((_B, 1), jnp.float32),
        jax.ShapeDtypeStruct((_B, 1), jnp.float32),
    ],
    scratch_shapes=[
        pltpu.VMEM((_B, 1), jnp.float32),
        pltpu.VMEM((_B, 1), jnp.float32),
        pltpu.VMEM((_B, 1), jnp.float32),
    ],
)


def _combine_body(m0_ref, s0_ref, t0_ref, m1_ref, s1_ref, t1_ref, out_ref):
    m0 = m0_ref[...]
    s0 = s0_ref[...]
    m1 = m1_ref[...]
    s1 = s1_ref[...]
    m = jnp.maximum(m0, jnp.max(m1, axis=1, keepdims=True))
    s = (s0 * jnp.exp2((m0 - m) * _K2)
         + jnp.sum(s1 * jnp.exp((m1 - m) * _SCALE), axis=1, keepdims=True))
    t = t0_ref[...] + jnp.sum(t1_ref[...], axis=1, keepdims=True)
    zt = (t - m) * _K2
    s_mod = s - jnp.exp2(zt) + jnp.exp2(zt - _SCALE * _MARGIN / _LN2)
    row_loss = (_SCALE * m + jnp.log(s_mod) - _SCALE * t
                + _SCALE * _MARGIN)
    out_ref[0, 0] = jnp.sum(row_loss) * (1.0 / _B)


_combine = pl.pallas_call(
    _combine_body,
    out_specs=pl.BlockSpec(memory_space=pltpu.SMEM),
    out_shape=jax.ShapeDtypeStruct((1, 1), jnp.float32),
)


def kernel(x, y):
    y32 = y.astype(jnp.int32)
    yrep = jnp.broadcast_to(y32[:, None], (_B, 128))
    m1, s1, t1 = _sc_region()(x, yrep)
    m0, s0, t0 = _tc_lse(y32.reshape(_B, 1), x)
    loss = _combine(m0, s0, t0, m1, s1, t1)
    return loss[0, 0]


# speedup vs baseline: 3.9432x; 1.2377x over previous
import functools

import jax
import jax.numpy as jnp
from jax import lax
from jax.experimental import pallas as pl
from jax.experimental.pallas import tpu as pltpu
from jax.experimental.pallas import tpu_sc as plsc

_SCALE = 30.0
_MARGIN = 0.4
_B = 1024
_C = 100000
_LN2 = 0.6931471805599453
_K2 = _SCALE / _LN2
_NEG = -1e30

_WSC = 196
_CSC = _WSC * 128
_WIN = 28
_NWIN = _WSC // _WIN
_WINC = _WIN * 128


_NC = 2
_NS = 16
_NW = _NC * _NS
_BANDS_PER_TEC = 128 // _NW


def _sc_body(x_hbm, yrep_hbm, m_hbm, s_hbm, t_hbm,
             xbuf, ytile, m_acc, s_acc, t_acc):
    wid = lax.axis_index("s") * _NC + lax.axis_index("c")
    iota = lax.broadcasted_iota(jnp.int32, (16,), 0)
    neg = jnp.full((16,), _NEG, jnp.float32)
    zero = jnp.zeros((16,), jnp.float32)

    def band_body(bi, _):
        r0 = pl.multiple_of((wid * _BANDS_PER_TEC + bi) * 8, 8)
        pltpu.sync_copy(yrep_hbm.at[pl.ds(r0, 8)], ytile)
        for r in range(8):
            for q in range(8):
                m_acc[r, pl.ds(q * 16, 16)] = neg
                s_acc[r, pl.ds(q * 16, 16)] = zero
                t_acc[r, pl.ds(q * 16, 16)] = zero

        def window(w, _):
            c0 = pl.multiple_of(w * _WINC, 128)
            pltpu.sync_copy(x_hbm.at[pl.ds(r0, 8), pl.ds(c0, _WINC)], xbuf)
            for r in range(8):
                yb = ytile[r, pl.ds(0, 16)]
                m_prev = tuple(m_acc[r, pl.ds(q * 16, 16)] for q in range(8))

                def pass1(tl, m):
                    return tuple(
                        jnp.maximum(m[q],
                                    xbuf[r, pl.ds(tl * 128 + q * 16, 16)])
                        for q in range(8))

                m_new = lax.fori_loop(0, _WIN, pass1, m_prev)
                s0 = tuple(
                    s_acc[r, pl.ds(q * 16, 16)]
                    * jnp.exp((m_prev[q] - m_new[q]) * _SCALE)
                    for q in range(8))
                t0 = tuple(t_acc[r, pl.ds(q * 16, 16)] for q in range(8))
                ydel = tuple(yb - (q * 16) - iota for q in range(8))

                def pass2(tl, st):
                    s, t = st
                    colb = c0 + tl * 128
                    vs = tuple(xbuf[r, pl.ds(tl * 128 + q * 16, 16)]
                               for q in range(8))
                    s = tuple(
                        s[q] + jnp.exp((vs[q] - m_new[q]) * _SCALE)
                        for q in range(8))
                    t = tuple(
                        t[q] + jnp.where(ydel[q] == colb, vs[q], 0.0)
                        for q in range(8))
                    return (s, t)

                s1, t1 = lax.fori_loop(0, _WIN, pass2, (s0, t0))
                for q in range(8):
                    m_acc[r, pl.ds(q * 16, 16)] = m_new[q]
                    s_acc[r, pl.ds(q * 16, 16)] = s1[q]
                    t_acc[r, pl.ds(q * 16, 16)] = t1[q]
            return 0

        lax.fori_loop(0, _NWIN, window, 0)

        pltpu.sync_copy(m_acc, m_hbm.at[pl.ds(r0, 8)])
        pltpu.sync_copy(s_acc, s_hbm.at[pl.ds(r0, 8)])
        pltpu.sync_copy(t_acc, t_hbm.at[pl.ds(r0, 8)])
        return 0

    lax.fori_loop(0, _BANDS_PER_TEC, band_body, 0)


@functools.cache
def _sc_region():
    mesh = plsc.VectorSubcoreMesh(
        core_axis_name="c", subcore_axis_name="s",
        num_cores=_NC, num_subcores=_NS)
    return pl.kernel(
        _sc_body,
        out_type=(
            jax.ShapeDtypeStruct((_B, 128), jnp.float32),
            jax.ShapeDtypeStruct((_B, 128), jnp.float32),
            jax.ShapeDtypeStruct((_B, 128), jnp.float32),
        ),
        mesh=mesh,
        scratch_types=[
            pltpu.VMEM((8, _WINC), jnp.float32),
            pltpu.VMEM((8, 128), jnp.int32),
            pltpu.VMEM((8, 128), jnp.float32),
            pltpu.VMEM((8, 128), jnp.float32),
            pltpu.VMEM((8, 128), jnp.float32),
        ],
        compiler_params=pltpu.CompilerParams(use_tc_tiling_on_sc=True),
    )


_BC = 3584
_K0 = _CSC // _BC
_K = (_C - _CSC + _BC - 1) // _BC


def _tc_body(y_ref, x_ref, m_out, s_out, t_out, m_ref, s_ref, t_ref):
    k = pl.program_id(0)
    nk = pl.num_programs(0)

    @pl.when(k == 0)
    def _init():
        m_ref[...] = jnp.full_like(m_ref, _NEG)
        s_ref[...] = jnp.zeros_like(s_ref)
        t_ref[...] = jnp.zeros_like(t_ref)

    def step(xb, col):
        y2 = y_ref[...]
        t_ref[...] += jnp.sum(jnp.where(col == y2, xb, 0.0), axis=1,
                              keepdims=True)
        m_old = m_ref[...]
        m_new = jnp.maximum(m_old, jnp.max(xb, axis=1, keepdims=True))
        s_ref[...] = (s_ref[...] * jnp.exp2((m_old - m_new) * _K2)
                      + jnp.sum(jnp.exp2((xb - m_new) * _K2), axis=1,
                                keepdims=True))
        m_ref[...] = m_new

    def colidx(k):
        return (_K0 + k) * _BC + lax.broadcasted_iota(jnp.int32, (_B, _BC), 1)

    @pl.when(k < nk - 1)
    def _hot():
        step(x_ref[...], colidx(k))

    @pl.when(k == nk - 1)
    def _tail():
        col = colidx(k)
        step(jnp.where(col < _C, x_ref[...], _NEG), col)
        m_out[...] = m_ref[...]
        s_out[...] = s_ref[...]
        t_out[...] = t_ref[...]


_tc_lse = pl.pallas_call(
    _tc_body,
    grid=(_K,),
    in_specs=[
        pl.BlockSpec((_B, 1), lambda k: (0, 0)),
        pl.BlockSpec((_B, _BC), lambda k: (0, _K0 + k)),
    ],
    out_specs=[
        pl.BlockSpec((_B, 1), lambda k: (0, 0)),
        pl.BlockSpec((_B, 1), lambda k: (0, 0)),
        pl.BlockSpec((_B, 1), lambda k: (0, 0)),
    ],
    out_shape=[
        jax.ShapeDtypeStruct((_B, 1), jnp.float32),
        jax.ShapeDtypeStruct((_B, 1), jnp.float32),
        jax.ShapeDtypeStruct((_B, 1), jnp.float32),
    ],
    scratch_shapes=[
        pltpu.VMEM((_B, 1), jnp.float32),
        pltpu.VMEM((_B, 1), jnp.float32),
        pltpu.VMEM((_B, 1), jnp.float32),
    ],
)


def _combine_body(m0_ref, s0_ref, t0_ref, m1_ref, s1_ref, t1_ref, out_ref):
    m0 = m0_ref[...]
    s0 = s0_ref[...]
    m1 = m1_ref[...]
    s1 = s1_ref[...]
    m = jnp.maximum(m0, jnp.max(m1, axis=1, keepdims=True))
    s = (s0 * jnp.exp2((m0 - m) * _K2)
         + jnp.sum(s1 * jnp.exp((m1 - m) * _SCALE), axis=1, keepdims=True))
    t = t0_ref[...] + jnp.sum(t1_ref[...], axis=1, keepdims=True)
    zt = (t - m) * _K2
    s_mod = s - jnp.exp2(zt) + jnp.exp2(zt - _SCALE * _MARGIN / _LN2)
    row_loss = (_SCALE * m + jnp.log(s_mod) - _SCALE * t
                + _SCALE * _MARGIN)
    out_ref[0, 0] = jnp.sum(row_loss) * (1.0 / _B)


_combine = pl.pallas_call(
    _combine_body,
    out_specs=pl.BlockSpec(memory_space=pltpu.SMEM),
    out_shape=jax.ShapeDtypeStruct((1, 1), jnp.float32),
)


def kernel(x, y):
    y32 = y.astype(jnp.int32)
    yrep = jnp.broadcast_to(y32[:, None], (_B, 128))
    m1, s1, t1 = _sc_region()(x, yrep)
    m0, s0, t0 = _tc_lse(y32.reshape(_B, 1), x)
    loss = _combine(m0, s0, t0, m1, s1, t1)
    return loss[0, 0]


# speedup vs baseline: 3.9438x; 1.0002x over previous
import functools

import jax
import jax.numpy as jnp
from jax import lax
from jax.experimental import pallas as pl
from jax.experimental.pallas import tpu as pltpu
from jax.experimental.pallas import tpu_sc as plsc

_SCALE = 30.0
_MARGIN = 0.4
_B = 1024
_C = 100000
_LN2 = 0.6931471805599453
_K2 = _SCALE / _LN2
_NEG = -1e30

_WSC = 196
_CSC = _WSC * 128
_WIN = 28
_NWIN = _WSC // _WIN
_WINC = _WIN * 128


_NC = 2
_NS = 16
_NW = _NC * _NS
_BANDS_PER_TEC = 128 // _NW


def _sc_body(x_hbm, yrep_hbm, m_hbm, s_hbm, t_hbm,
             xbuf, ytile, m_acc, s_acc, t_acc):
    wid = lax.axis_index("s") * _NC + lax.axis_index("c")
    iota = lax.broadcasted_iota(jnp.int32, (16,), 0)
    neg = jnp.full((16,), _NEG, jnp.float32)
    zero = jnp.zeros((16,), jnp.float32)

    def band_body(bi, _):
        r0 = pl.multiple_of((wid * _BANDS_PER_TEC + bi) * 8, 8)
        pltpu.sync_copy(yrep_hbm.at[pl.ds(r0, 8)], ytile)
        for r in range(8):
            for q in range(8):
                m_acc[r, pl.ds(q * 16, 16)] = neg
                s_acc[r, pl.ds(q * 16, 16)] = zero
                t_acc[r, pl.ds(q * 16, 16)] = zero

        def window(w, _):
            c0 = pl.multiple_of(w * _WINC, 128)
            pltpu.sync_copy(x_hbm.at[pl.ds(r0, 8), pl.ds(c0, _WINC)], xbuf)
            for r in range(8):
                yb = ytile[r, pl.ds(0, 16)]
                m_prev = tuple(m_acc[r, pl.ds(q * 16, 16)] for q in range(8))

                def pass1(tl, m):
                    return tuple(
                        jnp.maximum(m[q],
                                    xbuf[r, pl.ds(tl * 128 + q * 16, 16)])
                        for q in range(8))

                m_new = lax.fori_loop(0, _WIN, pass1, m_prev)
                s0 = tuple(
                    s_acc[r, pl.ds(q * 16, 16)]
                    * jnp.exp((m_prev[q] - m_new[q]) * _SCALE)
                    for q in range(8))
                t0 = tuple(t_acc[r, pl.ds(q * 16, 16)] for q in range(8))
                ydel = tuple(yb - (q * 16) - iota for q in range(8))

                def pass2(tl, st):
                    s, t = st
                    colb = c0 + tl * 128
                    vs = tuple(xbuf[r, pl.ds(tl * 128 + q * 16, 16)]
                               for q in range(8))
                    s = tuple(
                        s[q] + jnp.exp((vs[q] - m_new[q]) * _SCALE)
                        for q in range(8))
                    t = tuple(
                        t[q] + jnp.where(ydel[q] == colb, vs[q], 0.0)
                        for q in range(8))
                    return (s, t)

                s1, t1 = lax.fori_loop(0, _WIN, pass2, (s0, t0))
                for q in range(8):
                    m_acc[r, pl.ds(q * 16, 16)] = m_new[q]
                    s_acc[r, pl.ds(q * 16, 16)] = s1[q]
                    t_acc[r, pl.ds(q * 16, 16)] = t1[q]
            return 0

        lax.fori_loop(0, _NWIN, window, 0)

        pltpu.sync_copy(m_acc, m_hbm.at[pl.ds(r0, 8)])
        pltpu.sync_copy(s_acc, s_hbm.at[pl.ds(r0, 8)])
        pltpu.sync_copy(t_acc, t_hbm.at[pl.ds(r0, 8)])
        return 0

    lax.fori_loop(0, _BANDS_PER_TEC, band_body, 0)


@functools.cache
def _sc_region():
    mesh = plsc.VectorSubcoreMesh(
        core_axis_name="c", subcore_axis_name="s",
        num_cores=_NC, num_subcores=_NS)
    return pl.kernel(
        _sc_body,
        out_type=(
            jax.ShapeDtypeStruct((_B, 128), jnp.float32),
            jax.ShapeDtypeStruct((_B, 128), jnp.float32),
            jax.ShapeDtypeStruct((_B, 128), jnp.float32),
        ),
        mesh=mesh,
        scratch_types=[
            pltpu.VMEM((8, _WINC), jnp.float32),
            pltpu.VMEM((8, 128), jnp.int32),
            pltpu.VMEM((8, 128), jnp.float32),
            pltpu.VMEM((8, 128), jnp.float32),
            pltpu.VMEM((8, 128), jnp.float32),
        ],
        compiler_params=pltpu.CompilerParams(use_tc_tiling_on_sc=True),
    )


_BC = 3584
_K0 = _CSC // _BC
_K = (_C - _CSC + _BC - 1) // _BC


def _tc_body(y_ref, x_ref, m_out, s_out, t_out, m_ref, s_ref, t_ref):
    k = pl.program_id(0)
    nk = pl.num_programs(0)

    @pl.when(k == 0)
    def _init():
        m_ref[...] = jnp.full_like(m_ref, _NEG)
        s_ref[...] = jnp.zeros_like(s_ref)
        t_ref[...] = jnp.zeros_like(t_ref)

    def step(xb, col):
        y2 = y_ref[...]
        t_ref[...] += jnp.sum(jnp.where(col == y2, xb, 0.0), axis=1,
                              keepdims=True)
        m_old = m_ref[...]
        m_new = jnp.maximum(m_old, jnp.max(xb, axis=1, keepdims=True))
        s_ref[...] = (s_ref[...] * jnp.exp2((m_old - m_new) * _K2)
                      + jnp.sum(jnp.exp2((xb - m_new) * _K2), axis=1,
                                keepdims=True))
        m_ref[...] = m_new

    def colidx(k):
        return (_K0 + k) * _BC + lax.broadcasted_iota(jnp.int32, (_B, _BC), 1)

    @pl.when(k < nk - 1)
    def _hot():
        step(x_ref[...], colidx(k))

    @pl.when(k == nk - 1)
    def _tail():
        col = colidx(k)
        step(jnp.where(col < _C, x_ref[...], _NEG), col)
        m_out[...] = m_ref[...]
        s_out[...] = s_ref[...]
        t_out[...] = t_ref[...]


_tc_lse = pl.pallas_call(
    _tc_body,
    grid=(_K,),
    in_specs=[
        pl.BlockSpec((_B, 1), lambda k: (0, 0)),
        pl.BlockSpec((_B, _BC), lambda k: (0, _K0 + k)),
    ],
    out_specs=[
        pl.BlockSpec((_B, 1), lambda k: (0, 0)),
        pl.BlockSpec((_B, 1), lambda k: (0, 0)),
        pl.BlockSpec((_B, 1), lambda k: (0, 0)),
    ],
    out_shape=[
        jax.ShapeDtypeStruct((_B, 1), jnp.float32),
        jax.ShapeDtypeStruct((_B, 1), jnp.float32),
        jax.ShapeDtypeStruct((_B, 1), jnp.float32),
    ],
    scratch_shapes=[
        pltpu.VMEM((_B, 1), jnp.float32),
        pltpu.VMEM((_B, 1), jnp.float32),
        pltpu.VMEM((_B, 1), jnp.float32),
    ],
)


def _combine_body(m0_ref, s0_ref, t0_ref, m1_ref, s1_ref, t1_ref, out_ref):
    m0 = m0_ref[...]
    s0 = s0_ref[...]
    m1 = m1_ref[...]
    s1 = s1_ref[...]
    m = jnp.maximum(m0, jnp.max(m1, axis=1, keepdims=True))
    s = (s0 * jnp.exp2((m0 - m) * _K2)
         + jnp.sum(s1 * jnp.exp((m1 - m) * _SCALE), axis=1, keepdims=True))
    t = t0_ref[...] + jnp.sum(t1_ref[...], axis=1, keepdims=True)
    zt = (t - m) * _K2
    s_mod = s - jnp.exp2(zt) + jnp.exp2(zt - _SCALE * _MARGIN / _LN2)
    row_loss = (_SCALE * m + jnp.log(s_mod) - _SCALE * t
                + _SCALE * _MARGIN)
    out_ref[0, 0] = jnp.sum(row_loss) * (1.0 / _B)


_combine = pl.pallas_call(
    _combine_body,
    out_specs=pl.BlockSpec(memory_space=pltpu.SMEM),
    out_shape=jax.ShapeDtypeStruct((1, 1), jnp.float32),
)


def kernel(x, y):
    y32 = y.astype(jnp.int32)
    yrep = jnp.broadcast_to(y32[:, None], (_B, 128))
    m0, s0, t0 = _tc_lse(y32.reshape(_B, 1), x)
    m1, s1, t1 = _sc_region()(x, yrep)
    loss = _combine(m0, s0, t0, m1, s1, t1)
    return loss[0, 0]


# speedup vs baseline: 4.2521x; 1.0782x over previous
import functools

import jax
import jax.numpy as jnp
from jax import lax
from jax.experimental import pallas as pl
from jax.experimental.pallas import tpu as pltpu
from jax.experimental.pallas import tpu_sc as plsc

_SCALE = 30.0
_MARGIN = 0.4
_B = 1024
_C = 100000
_LN2 = 0.6931471805599453
_K2 = _SCALE / _LN2
_NEG = -1e30

_WSC = 56
_CSC = _WSC * 128
_WIN = 28
_NWIN = _WSC // _WIN
_WINC = _WIN * 128


_NC = 2
_NS = 16
_NW = _NC * _NS
_BANDS_PER_TEC = 128 // _NW


def _sc_body(x_hbm, yrep_hbm, m_hbm, s_hbm, t_hbm,
             xbuf, ytile, m_acc, s_acc, t_acc):
    wid = lax.axis_index("s") * _NC + lax.axis_index("c")
    iota = lax.broadcasted_iota(jnp.int32, (16,), 0)
    neg = jnp.full((16,), _NEG, jnp.float32)
    zero = jnp.zeros((16,), jnp.float32)

    def band_body(bi, _):
        r0 = pl.multiple_of((wid * _BANDS_PER_TEC + bi) * 8, 8)
        pltpu.sync_copy(yrep_hbm.at[pl.ds(r0, 8)], ytile)
        for r in range(8):
            for q in range(8):
                m_acc[r, pl.ds(q * 16, 16)] = neg
                s_acc[r, pl.ds(q * 16, 16)] = zero
                t_acc[r, pl.ds(q * 16, 16)] = zero

        def window(w, _):
            c0 = pl.multiple_of(w * _WINC, 128)
            pltpu.sync_copy(x_hbm.at[pl.ds(r0, 8), pl.ds(c0, _WINC)], xbuf)
            for r in range(8):
                yb = ytile[r, pl.ds(0, 16)]
                m_prev = tuple(m_acc[r, pl.ds(q * 16, 16)] for q in range(8))

                def pass1(tl, m):
                    return tuple(
                        jnp.maximum(m[q],
                                    xbuf[r, pl.ds(tl * 128 + q * 16, 16)])
                        for q in range(8))

                m_new = lax.fori_loop(0, _WIN, pass1, m_prev)
                s0 = tuple(
                    s_acc[r, pl.ds(q * 16, 16)]
                    * jnp.exp((m_prev[q] - m_new[q]) * _SCALE)
                    for q in range(8))
                t0 = tuple(t_acc[r, pl.ds(q * 16, 16)] for q in range(8))
                ydel = tuple(yb - (q * 16) - iota for q in range(8))

                def pass2(tl, st):
                    s, t = st
                    colb = c0 + tl * 128
                    vs = tuple(xbuf[r, pl.ds(tl * 128 + q * 16, 16)]
                               for q in range(8))
                    s = tuple(
                        s[q] + jnp.exp((vs[q] - m_new[q]) * _SCALE)
                        for q in range(8))
                    t = tuple(
                        t[q] + jnp.where(ydel[q] == colb, vs[q], 0.0)
                        for q in range(8))
                    return (s, t)

                s1, t1 = lax.fori_loop(0, _WIN, pass2, (s0, t0))
                for q in range(8):
                    m_acc[r, pl.ds(q * 16, 16)] = m_new[q]
                    s_acc[r, pl.ds(q * 16, 16)] = s1[q]
                    t_acc[r, pl.ds(q * 16, 16)] = t1[q]
            return 0

        lax.fori_loop(0, _NWIN, window, 0)

        pltpu.sync_copy(m_acc, m_hbm.at[pl.ds(r0, 8)])
        pltpu.sync_copy(s_acc, s_hbm.at[pl.ds(r0, 8)])
        pltpu.sync_copy(t_acc, t_hbm.at[pl.ds(r0, 8)])
        return 0

    lax.fori_loop(0, _BANDS_PER_TEC, band_body, 0)


@functools.cache
def _sc_region():
    mesh = plsc.VectorSubcoreMesh(
        core_axis_name="c", subcore_axis_name="s",
        num_cores=_NC, num_subcores=_NS)
    return pl.kernel(
        _sc_body,
        out_type=(
            jax.ShapeDtypeStruct((_B, 128), jnp.float32),
            jax.ShapeDtypeStruct((_B, 128), jnp.float32),
            jax.ShapeDtypeStruct((_B, 128), jnp.float32),
        ),
        mesh=mesh,
        scratch_types=[
            pltpu.VMEM((8, _WINC), jnp.float32),
            pltpu.VMEM((8, 128), jnp.int32),
            pltpu.VMEM((8, 128), jnp.float32),
            pltpu.VMEM((8, 128), jnp.float32),
            pltpu.VMEM((8, 128), jnp.float32),
        ],
        compiler_params=pltpu.CompilerParams(use_tc_tiling_on_sc=True),
    )


_BC = 3584
_K0 = _CSC // _BC
_K = (_C - _CSC + _BC - 1) // _BC


def _tc_body(y_ref, x_ref, m_out, s_out, t_out, m_ref, s_ref, t_ref):
    k = pl.program_id(0)
    nk = pl.num_programs(0)

    @pl.when(k == 0)
    def _init():
        m_ref[...] = jnp.full_like(m_ref, _NEG)
        s_ref[...] = jnp.zeros_like(s_ref)
        t_ref[...] = jnp.zeros_like(t_ref)

    def step(xb, col):
        y2 = y_ref[...]
        t_ref[...] += jnp.sum(jnp.where(col == y2, xb, 0.0), axis=1,
                              keepdims=True)
        m_old = m_ref[...]
        m_new = jnp.maximum(m_old, jnp.max(xb, axis=1, keepdims=True))
        s_ref[...] = (s_ref[...] * jnp.exp2((m_old - m_new) * _K2)
                      + jnp.sum(jnp.exp2((xb - m_new) * _K2), axis=1,
                                keepdims=True))
        m_ref[...] = m_new

    def colidx(k):
        return (_K0 + k) * _BC + lax.broadcasted_iota(jnp.int32, (_B, _BC), 1)

    @pl.when(k < nk - 1)
    def _hot():
        step(x_ref[...], colidx(k))

    @pl.when(k == nk - 1)
    def _tail():
        col = colidx(k)
        step(jnp.where(col < _C, x_ref[...], _NEG), col)
        m_out[...] = m_ref[...]
        s_out[...] = s_ref[...]
        t_out[...] = t_ref[...]


_tc_lse = pl.pallas_call(
    _tc_body,
    grid=(_K,),
    in_specs=[
        pl.BlockSpec((_B, 1), lambda k: (0, 0)),
        pl.BlockSpec((_B, _BC), lambda k: (0, _K0 + k)),
    ],
    out_specs=[
        pl.BlockSpec((_B, 1), lambda k: (0, 0)),
        pl.BlockSpec((_B, 1), lambda k: (0, 0)),
        pl.BlockSpec((_B, 1), lambda k: (0, 0)),
    ],
    out_shape=[
        jax.ShapeDtypeStruct((_B, 1), jnp.float32),
        jax.ShapeDtypeStruct((_B, 1), jnp.float32),
        jax.ShapeDtypeStruct((_B, 1), jnp.float32),
    ],
    scratch_shapes=[
        pltpu.VMEM((_B, 1), jnp.float32),
        pltpu.VMEM((_B, 1), jnp.float32),
        pltpu.VMEM((_B, 1), jnp.float32),
    ],
)


def _combine_body(m0_ref, s0_ref, t0_ref, m1_ref, s1_ref, t1_ref, out_ref):
    m0 = m0_ref[...]
    s0 = s0_ref[...]
    m1 = m1_ref[...]
    s1 = s1_ref[...]
    m = jnp.maximum(m0, jnp.max(m1, axis=1, keepdims=True))
    s = (s0 * jnp.exp2((m0 - m) * _K2)
         + jnp.sum(s1 * jnp.exp((m1 - m) * _SCALE), axis=1, keepdims=True))
    t = t0_ref[...] + jnp.sum(t1_ref[...], axis=1, keepdims=True)
    zt = (t - m) * _K2
    s_mod = s - jnp.exp2(zt) + jnp.exp2(zt - _SCALE * _MARGIN / _LN2)
    row_loss = (_SCALE * m + jnp.log(s_mod) - _SCALE * t
                + _SCALE * _MARGIN)
    out_ref[0, 0] = jnp.sum(row_loss) * (1.0 / _B)


_combine = pl.pallas_call(
    _combine_body,
    out_specs=pl.BlockSpec(memory_space=pltpu.SMEM),
    out_shape=jax.ShapeDtypeStruct((1, 1), jnp.float32),
)


def kernel(x, y):
    y32 = y.astype(jnp.int32)
    yrep = jnp.broadcast_to(y32[:, None], (_B, 128))
    m0, s0, t0 = _tc_lse(y32.reshape(_B, 1), x)
    m1, s1, t1 = _sc_region()(x, yrep)
    loss = _combine(m0, s0, t0, m1, s1, t1)
    return loss[0, 0]
